# baseline (speedup 1.0000x reference)
import functools

import jax
import jax.numpy as jnp
from jax import lax
from jax.experimental import pallas as pl
from jax.experimental.pallas import tpu as pltpu
from jax.experimental.pallas import tpu_sc as plsc

_F32_MIN = -3.4028235e38


def _norm_kernel(q_ref, qr_ref):
    x = q_ref[...]
    ss = jnp.sum(x * x, axis=1, keepdims=True)
    inv = 1.0 / jnp.maximum(jnp.sqrt(ss), 1e-12)
    y = x * inv
    bs, c, t, d = x.shape
    qr_ref[...] = y.reshape(bs * c * t, d)


def _stats_kernel(q_ref, k_ref, m1_ref, cm_ref, cs_ref, a1_ref, a2_ref, g_ref):
    i = pl.program_id(0)
    qi = q_ref[...]
    kk = k_ref[...]
    s = jax.lax.dot_general(qi, kk, (((1,), (1,)), ((), ())),
                            preferred_element_type=jnp.float32)
    bns, m = s.shape
    m1 = jnp.max(s, axis=1, keepdims=True)
    m1_ref[...] = m1
    g_ref[...] = jnp.concatenate(
        [jnp.exp(m1) * qi, jnp.zeros_like(qi)], axis=1)

    iota_col = jax.lax.broadcasted_iota(
        jnp.int32, (m, 1), 0).astype(jnp.float32)
    oh1b = s == m1
    oh1 = oh1b.astype(jnp.float32)
    a1f = jnp.dot(oh1, iota_col, preferred_element_type=jnp.float32)
    masked = jnp.where(oh1b, _F32_MIN, s)
    m2 = jnp.max(masked, axis=1, keepdims=True)
    oh2 = (masked == m2).astype(jnp.float32)
    a2f = jnp.dot(oh2, iota_col, preferred_element_type=jnp.float32)
    a1_ref[...] = a1f.astype(jnp.int32)
    a2_ref[...] = a2f.astype(jnp.int32)

    @pl.when(i == 0)
    def _():
        cm_ref[...] = jnp.full_like(cm_ref, _F32_MIN)
        cs_ref[...] = jnp.zeros_like(cs_ref)

    cm = cm_ref[...]
    cs = cs_ref[...]
    bm = jnp.max(s, axis=0)[None, :]
    ncm = jnp.maximum(cm, bm)
    cs = cs * jnp.exp(cm - ncm) + jnp.sum(jnp.exp(s - ncm), axis=0)[None, :]
    cm_ref[...] = ncm
    cs_ref[...] = cs


def _emit_kernel(q_ref, k_ref, m1_ref, cm_ref, cs_ref, sq_ref, sm_ref, uq_ref):
    qi = q_ref[...]
    kk = k_ref[...]
    s = jax.lax.dot_general(qi, kk, (((1,), (1,)), ((), ())),
                            preferred_element_type=jnp.float32)
    m1 = m1_ref[...]
    cm = cm_ref[...]
    csinv = 1.0 / cs_ref[...]
    e1 = jnp.exp(s - m1)
    rsinv = 1.0 / jnp.sum(e1, axis=1, keepdims=True)
    pm = e1 * rsinv
    sm_ref[...] = pm
    sq_ref[...] = jnp.exp(s - cm) * csinv
    uq_ref[...] = jnp.dot(pm, kk, preferred_element_type=jnp.float32)


def _epi_kernel(qu2_ref, cmt_ref, k_ref, pos_ref, neg_ref, qr_ref, uq_ref,
                um_ref, sl_ref, cl_ref, *, n_total):
    kk = k_ref[...]
    d = kk.shape[1]
    qu2 = qu2_ref[...]
    qu = (qu2[0, :, :d] + qu2[1, :, :d]) * jnp.exp(-cmt_ref[...])
    um = qu + kk
    nrm = jnp.maximum(jnp.sqrt(jnp.sum(um * um, axis=1, keepdims=True)), 1e-12)
    um_ref[...] = um / nrm

    qr = qr_ref[...]
    pos = pos_ref[...][:, :d]
    neg = neg_ref[...][:, :d]
    dpp = qr - pos
    cl_ref[...] = (jnp.sum(dpp * dpp) / (n_total * kk.shape[1]))[None, None]
    dp = jnp.sqrt(jnp.sum((dpp + 1e-6) ** 2, axis=1))
    dnn = jnp.sqrt(jnp.sum((qr - neg + 1e-6) ** 2, axis=1))
    sl = jnp.sum(jnp.maximum(dp - dnn + 1.0, 0.0)) / n_total
    sl_ref[...] = (sl + 0.0 * uq_ref[0, 0])[None, None]


def _sc_gather_scatter(n, m, dp):
    rows_per_worker = n // 32
    chunk = 128
    nchunks = rows_per_worker // chunk
    mesh = plsc.VectorSubcoreMesh(core_axis_name="c", subcore_axis_name="s")
    f32 = jnp.float32

    @functools.partial(
        pl.kernel,
        mesh=mesh,
        out_type=[jax.ShapeDtypeStruct((n, dp), f32),
                  jax.ShapeDtypeStruct((n, dp), f32),
                  jax.ShapeDtypeStruct((2 * m, dp), f32)],
        scratch_types=[pltpu.VMEM((nchunks, chunk), jnp.int32),
                       pltpu.VMEM((nchunks, chunk), jnp.int32),
                       pltpu.VMEM((chunk, dp), f32),
                       pltpu.VMEM((chunk, dp), f32),
                       pltpu.VMEM((nchunks, chunk, dp), f32),
                       pltpu.VMEM_SHARED((m, dp), f32),
                       pltpu.SemaphoreType.DMA,
                       pltpu.SemaphoreType.DMA,
                       pltpu.SemaphoreType.DMA,
                       pltpu.SemaphoreType.DMA],
    )
    def sc_kernel(k_hbm, a1_hbm, a2_hbm, g_hbm, z_hbm,
                  pos_hbm, neg_hbm, qu2_hbm,
                  idx1_v, idx2_v, pos_v, neg_v, g_v, acc_sh,
                  sem_ga, sem_g, sem_st, sem_z):
        cid = lax.axis_index("c")
        sid = lax.axis_index("s")
        wid = sid * 2 + cid
        base = wid * rows_per_worker
        mslice = m // 16

        zinit = pltpu.async_copy(z_hbm.at[pl.ds(sid * mslice, mslice)],
                                 acc_sh.at[pl.ds(sid * mslice, mslice)], sem_z)

        gloads = []
        for ch in range(nchunks):
            off = base + ch * chunk
            pltpu.sync_copy(a1_hbm.at[pl.ds(off, chunk)], idx1_v.at[ch])
            pltpu.sync_copy(a2_hbm.at[pl.ds(off, chunk)], idx2_v.at[ch])
            gloads.append(pltpu.async_copy(g_hbm.at[pl.ds(off, chunk)],
                                           g_v.at[ch], sem_g))

        stores = []
        for ch in range(nchunks):
            off = base + ch * chunk
            hp = pltpu.async_copy(k_hbm.at[idx1_v.at[ch]], pos_v, sem_ga)
            hn = pltpu.async_copy(k_hbm.at[idx2_v.at[ch]], neg_v, sem_ga)
            hp.wait()
            hn.wait()
            sp = pltpu.async_copy(pos_v, pos_hbm.at[pl.ds(off, chunk)], sem_st)
            sn = pltpu.async_copy(neg_v, neg_hbm.at[pl.ds(off, chunk)], sem_st)
            if ch < nchunks - 1:
                sp.wait()
                sn.wait()
            else:
                stores += [sp, sn]

        for h in gloads:
            h.wait()
        zinit.wait()
        plsc.subcore_barrier()

        for ch in range(nchunks):
            pltpu.sync_copy(g_v.at[ch], acc_sh.at[idx1_v.at[ch]], add=True)
        for h in stores:
            h.wait()

        plsc.subcore_barrier()
        pltpu.sync_copy(acc_sh.at[pl.ds(sid * mslice, mslice)],
                        qu2_hbm.at[pl.ds(cid * m + sid * mslice, mslice)])

    return sc_kernel


def kernel(query, keys):
    bs, c, t, d = query.shape
    m = keys.shape[0]
    n = bs * c * t
    bns = 1024
    bn = 256
    f32 = jnp.float32

    qr = pl.pallas_call(
        _norm_kernel,
        out_shape=jax.ShapeDtypeStruct((n, d), f32),
    )(query)

    col_spec = pl.BlockSpec((1, m), lambda i: (0, 0))
    k_spec = pl.BlockSpec((m, d), lambda i: (0, 0))

    m1, cm, cs, a1, a2, g = pl.pallas_call(
        _stats_kernel,
        grid=(n // bns,),
        in_specs=[pl.BlockSpec((bns, d), lambda i: (i, 0)), k_spec],
        out_specs=[pl.BlockSpec((bns, 1), lambda i: (i, 0)),
                   col_spec, col_spec,
                   pl.BlockSpec((bns, 1), lambda i: (i, 0)),
                   pl.BlockSpec((bns, 1), lambda i: (i, 0)),
                   pl.BlockSpec((bns, 2 * d), lambda i: (i, 0))],
        out_shape=[jax.ShapeDtypeStruct((n, 1), f32),
                   jax.ShapeDtypeStruct((1, m), f32),
                   jax.ShapeDtypeStruct((1, m), f32),
                   jax.ShapeDtypeStruct((n, 1), jnp.int32),
                   jax.ShapeDtypeStruct((n, 1), jnp.int32),
                   jax.ShapeDtypeStruct((n, 2 * d), f32)],
    )(qr, keys)

    keys_pad = jnp.concatenate([keys, jnp.zeros_like(keys)], axis=1)
    zeros = jnp.zeros((m, 2 * d), f32)
    pos, neg, qu2 = _sc_gather_scatter(n, m, 2 * d)(
        keys_pad, a1.reshape(n), a2.reshape(n), g, zeros)

    sq, sm, uq = pl.pallas_call(
        _emit_kernel,
        grid=(n // bn,),
        in_specs=[pl.BlockSpec((bn, d), lambda i: (i, 0)), k_spec,
                  pl.BlockSpec((bn, 1), lambda i: (i, 0)),
                  col_spec, col_spec],
        out_specs=[pl.BlockSpec((bn, m), lambda i: (i, 0)),
                   pl.BlockSpec((bn, m), lambda i: (i, 0)),
                   pl.BlockSpec((bn, d), lambda i: (i, 0))],
        out_shape=[jax.ShapeDtypeStruct((n, m), f32),
                   jax.ShapeDtypeStruct((n, m), f32),
                   jax.ShapeDtypeStruct((n, d), f32)],
    )(qr, keys, m1, cm, cs)

    um, sl, cl = pl.pallas_call(
        functools.partial(_epi_kernel, n_total=n),
        out_shape=[jax.ShapeDtypeStruct((m, d), f32),
                   jax.ShapeDtypeStruct((1, 1), f32),
                   jax.ShapeDtypeStruct((1, 1), f32)],
    )(qu2.reshape(2, m, 2 * d), cm.reshape(m, 1), keys, pos, neg, qr, uq)

    updated_query = uq.reshape(bs, c, t, d)
    return (updated_query, um, sq, sm, sl.reshape(()), cl.reshape(()))


# speedup vs baseline: 1.0581x; 1.0581x over previous
import functools

import jax
import jax.numpy as jnp
from jax import lax
from jax.experimental import pallas as pl
from jax.experimental.pallas import tpu as pltpu
from jax.experimental.pallas import tpu_sc as plsc

_F32_MIN = -3.4028235e38


def _norm_kernel(q_ref, qr_ref):
    x = q_ref[...]
    ss = jnp.sum(x * x, axis=1, keepdims=True)
    inv = 1.0 / jnp.maximum(jnp.sqrt(ss), 1e-12)
    y = x * inv
    bs, c, t, d = x.shape
    qr_ref[...] = y.reshape(bs * c * t, d)


def _stats_kernel(q_ref, k_ref, m1_ref, cm_ref, cs_ref, a1_ref, a2_ref, g_ref):
    i = pl.program_id(0)
    qi = q_ref[...]
    kk = k_ref[...]
    s = jax.lax.dot_general(qi, kk, (((1,), (1,)), ((), ())),
                            preferred_element_type=jnp.float32)
    bns, m = s.shape
    m1 = jnp.max(s, axis=1, keepdims=True)
    m1_ref[...] = m1
    g_ref[...] = jnp.concatenate(
        [jnp.exp(m1) * qi, jnp.zeros_like(qi)], axis=1)

    iota = jax.lax.broadcasted_iota(jnp.int32, (bns, m), 1)
    oh1b = s == m1
    a1 = jnp.min(jnp.where(oh1b, iota, m), axis=1)
    masked = jnp.where(oh1b, _F32_MIN, s)
    m2 = jnp.max(masked, axis=1, keepdims=True)
    a2 = jnp.min(jnp.where(masked == m2, iota, m), axis=1)
    a1_ref[...] = a1[:, None]
    a2_ref[...] = a2[:, None]

    @pl.when(i == 0)
    def _():
        cm_ref[...] = jnp.full_like(cm_ref, _F32_MIN)
        cs_ref[...] = jnp.zeros_like(cs_ref)

    cm = cm_ref[...]
    cs = cs_ref[...]
    bm = jnp.max(s, axis=0)[None, :]
    ncm = jnp.maximum(cm, bm)
    cs = cs * jnp.exp(cm - ncm) + jnp.sum(jnp.exp(s - ncm), axis=0)[None, :]
    cm_ref[...] = ncm
    cs_ref[...] = cs


def _emit_kernel(q_ref, k_ref, m1_ref, cm_ref, cs_ref, sq_ref, sm_ref, uq_ref):
    qi = q_ref[...]
    kk = k_ref[...]
    s = jax.lax.dot_general(qi, kk, (((1,), (1,)), ((), ())),
                            preferred_element_type=jnp.float32)
    m1 = m1_ref[...]
    cm = cm_ref[...]
    csinv = 1.0 / cs_ref[...]
    e1 = jnp.exp(s - m1)
    rsinv = 1.0 / jnp.sum(e1, axis=1, keepdims=True)
    pm = e1 * rsinv
    sm_ref[...] = pm
    sq_ref[...] = jnp.exp(s - cm) * csinv
    uq_ref[...] = jnp.dot(pm, kk, preferred_element_type=jnp.float32)


def _epi_kernel(qu2_ref, cmt_ref, k_ref, pos_ref, neg_ref, qr_ref, uq_ref,
                um_ref, sl_ref, cl_ref, *, n_total):
    kk = k_ref[...]
    d = kk.shape[1]
    qu2 = qu2_ref[...]
    qu = (qu2[0, :, :d] + qu2[1, :, :d]) * jnp.exp(-cmt_ref[...])
    um = qu + kk
    nrm = jnp.maximum(jnp.sqrt(jnp.sum(um * um, axis=1, keepdims=True)), 1e-12)
    um_ref[...] = um / nrm

    qr = qr_ref[...]
    pos = pos_ref[...][:, :d]
    neg = neg_ref[...][:, :d]
    dpp = qr - pos
    cl_ref[...] = (jnp.sum(dpp * dpp) / (n_total * kk.shape[1]))[None, None]
    dp = jnp.sqrt(jnp.sum((dpp + 1e-6) ** 2, axis=1))
    dnn = jnp.sqrt(jnp.sum((qr - neg + 1e-6) ** 2, axis=1))
    sl = jnp.sum(jnp.maximum(dp - dnn + 1.0, 0.0)) / n_total
    sl_ref[...] = (sl + 0.0 * uq_ref[0, 0])[None, None]


def _sc_gather_scatter(n, m, dp):
    rows_per_worker = n // 32
    chunk = 128
    nchunks = rows_per_worker // chunk
    mesh = plsc.VectorSubcoreMesh(core_axis_name="c", subcore_axis_name="s")
    f32 = jnp.float32

    @functools.partial(
        pl.kernel,
        mesh=mesh,
        out_type=[jax.ShapeDtypeStruct((n, dp), f32),
                  jax.ShapeDtypeStruct((n, dp), f32),
                  jax.ShapeDtypeStruct((2 * m, dp), f32)],
        scratch_types=[pltpu.VMEM((nchunks, chunk), jnp.int32),
                       pltpu.VMEM((nchunks, chunk), jnp.int32),
                       pltpu.VMEM((chunk, dp), f32),
                       pltpu.VMEM((chunk, dp), f32),
                       pltpu.VMEM((nchunks, chunk, dp), f32),
                       pltpu.VMEM_SHARED((m, dp), f32),
                       pltpu.SemaphoreType.DMA,
                       pltpu.SemaphoreType.DMA,
                       pltpu.SemaphoreType.DMA,
                       pltpu.SemaphoreType.DMA,
                       pltpu.SemaphoreType.DMA,
                       pltpu.SemaphoreType.DMA],
    )
    def sc_kernel(k_hbm, a1_hbm, a2_hbm, g_hbm, z_hbm,
                  pos_hbm, neg_hbm, qu2_hbm,
                  idx1_v, idx2_v, pos_v, neg_v, g_v, acc_sh,
                  sem_gp, sem_gn, sem_sp, sem_sn, sem_g, sem_z):
        cid = lax.axis_index("c")
        sid = lax.axis_index("s")
        wid = sid * 2 + cid
        base = wid * rows_per_worker
        mslice = m // 16

        zinit = pltpu.async_copy(z_hbm.at[pl.ds(sid * mslice, mslice)],
                                 acc_sh.at[pl.ds(sid * mslice, mslice)], sem_z)

        gloads = []
        for ch in range(nchunks):
            off = base + ch * chunk
            pltpu.sync_copy(a1_hbm.at[pl.ds(off, chunk)], idx1_v.at[ch])
            pltpu.sync_copy(a2_hbm.at[pl.ds(off, chunk)], idx2_v.at[ch])
            gloads.append(pltpu.async_copy(g_hbm.at[pl.ds(off, chunk)],
                                           g_v.at[ch], sem_g))

        stores = []
        for ch in range(nchunks):
            off = base + ch * chunk
            hp = pltpu.async_copy(k_hbm.at[idx1_v.at[ch]], pos_v, sem_gp)
            hn = pltpu.async_copy(k_hbm.at[idx2_v.at[ch]], neg_v, sem_gn)
            hp.wait()
            hn.wait()
            sp = pltpu.async_copy(pos_v, pos_hbm.at[pl.ds(off, chunk)], sem_sp)
            sn = pltpu.async_copy(neg_v, neg_hbm.at[pl.ds(off, chunk)], sem_sn)
            if ch < nchunks - 1:
                sp.wait()
                sn.wait()
            else:
                stores += [sp, sn]

        for h in gloads:
            h.wait()
        zinit.wait()
        plsc.subcore_barrier()

        for ch in range(nchunks):
            pltpu.sync_copy(g_v.at[ch], acc_sh.at[idx1_v.at[ch]], add=True)
        for h in stores:
            h.wait()

        plsc.subcore_barrier()
        pltpu.sync_copy(acc_sh.at[pl.ds(sid * mslice, mslice)],
                        qu2_hbm.at[pl.ds(cid * m + sid * mslice, mslice)])

    return sc_kernel


def kernel(query, keys):
    bs, c, t, d = query.shape
    m = keys.shape[0]
    n = bs * c * t
    bns = 1024
    bn = 512
    f32 = jnp.float32

    qr = pl.pallas_call(
        _norm_kernel,
        out_shape=jax.ShapeDtypeStruct((n, d), f32),
    )(query)

    col_spec = pl.BlockSpec((1, m), lambda i: (0, 0))
    k_spec = pl.BlockSpec((m, d), lambda i: (0, 0))

    m1, cm, cs, a1, a2, g = pl.pallas_call(
        _stats_kernel,
        grid=(n // bns,),
        in_specs=[pl.BlockSpec((bns, d), lambda i: (i, 0)), k_spec],
        out_specs=[pl.BlockSpec((bns, 1), lambda i: (i, 0)),
                   col_spec, col_spec,
                   pl.BlockSpec((bns, 1), lambda i: (i, 0)),
                   pl.BlockSpec((bns, 1), lambda i: (i, 0)),
                   pl.BlockSpec((bns, 2 * d), lambda i: (i, 0))],
        out_shape=[jax.ShapeDtypeStruct((n, 1), f32),
                   jax.ShapeDtypeStruct((1, m), f32),
                   jax.ShapeDtypeStruct((1, m), f32),
                   jax.ShapeDtypeStruct((n, 1), jnp.int32),
                   jax.ShapeDtypeStruct((n, 1), jnp.int32),
                   jax.ShapeDtypeStruct((n, 2 * d), f32)],
    )(qr, keys)

    keys_pad = jnp.concatenate([keys, jnp.zeros_like(keys)], axis=1)
    zeros = jnp.zeros((m, 2 * d), f32)
    pos, neg, qu2 = _sc_gather_scatter(n, m, 2 * d)(
        keys_pad, a1.reshape(n), a2.reshape(n), g, zeros)

    sq, sm, uq = pl.pallas_call(
        _emit_kernel,
        grid=(n // bn,),
        in_specs=[pl.BlockSpec((bn, d), lambda i: (i, 0)), k_spec,
                  pl.BlockSpec((bn, 1), lambda i: (i, 0)),
                  col_spec, col_spec],
        out_specs=[pl.BlockSpec((bn, m), lambda i: (i, 0)),
                   pl.BlockSpec((bn, m), lambda i: (i, 0)),
                   pl.BlockSpec((bn, d), lambda i: (i, 0))],
        out_shape=[jax.ShapeDtypeStruct((n, m), f32),
                   jax.ShapeDtypeStruct((n, m), f32),
                   jax.ShapeDtypeStruct((n, d), f32)],
    )(qr, keys, m1, cm, cs)

    um, sl, cl = pl.pallas_call(
        functools.partial(_epi_kernel, n_total=n),
        out_shape=[jax.ShapeDtypeStruct((m, d), f32),
                   jax.ShapeDtypeStruct((1, 1), f32),
                   jax.ShapeDtypeStruct((1, 1), f32)],
    )(qu2.reshape(2, m, 2 * d), cm.reshape(m, 1), keys, pos, neg, qr, uq)

    updated_query = uq.reshape(bs, c, t, d)
    return (updated_query, um, sq, sm, sl.reshape(()), cl.reshape(()))
